# baseline (speedup 1.0000x reference)
import functools

import jax
import jax.numpy as jnp
from jax import lax
from jax.experimental import pallas as pl
from jax.experimental.pallas import tpu as pltpu
from jax.experimental.pallas import tpu_sc as plsc

_CP = 128


def _transpose_to_rows(x):
    B, C, H, W = x.shape
    HB = 16

    def body(x_ref, o_ref):
        xb = x_ref[0]
        xb = xb.reshape(C, HB * W)
        o_ref[...] = jnp.pad(xb.T, ((0, 0), (0, _CP - C)))

    return pl.pallas_call(
        body,
        grid=(B, H // HB),
        in_specs=[pl.BlockSpec((1, C, HB, W), lambda b, h: (b, 0, h, 0))],
        out_specs=pl.BlockSpec((HB * W, _CP), lambda b, h: (b * (H // HB) + h, 0)),
        out_shape=jax.ShapeDtypeStruct((B * H * W, _CP), x.dtype),
    )(x)


def _sc_sample(xt, posx, posy, rowb, B, C, H, W, N, NP):
    info = plsc.get_sparse_core_info()
    NC, NS, L = info.num_cores, info.num_subcores, info.num_lanes
    NW = NS
    PPW = NP // NW
    G = 128
    NCH = PPW // G
    CB = C // L

    mesh = plsc.VectorSubcoreMesh(core_axis_name="c", subcore_axis_name="s", num_cores=1)

    @functools.partial(
        pl.kernel, mesh=mesh,
        out_type=jax.ShapeDtypeStruct((NP, C), jnp.float32),
        scratch_types=[
            pltpu.VMEM((PPW,), jnp.float32),
            pltpu.VMEM((PPW,), jnp.float32),
            pltpu.VMEM((PPW,), jnp.int32),
            pltpu.VMEM((G,), jnp.int32),
            pltpu.VMEM((G,), jnp.int32),
            pltpu.VMEM((G,), jnp.int32),
            pltpu.VMEM((G,), jnp.int32),
            pltpu.VMEM((4, G), jnp.float32),
            pltpu.VMEM((G, _CP), jnp.float32),
            pltpu.VMEM((G, _CP), jnp.float32),
            pltpu.VMEM((G, _CP), jnp.float32),
            pltpu.VMEM((G, _CP), jnp.float32),
            pltpu.VMEM((G, C), jnp.float32),
            pltpu.SemaphoreType.DMA,
            pltpu.SemaphoreType.DMA,
            pltpu.SemaphoreType.DMA,
            pltpu.SemaphoreType.DMA,
        ],
    )
    def body(xt_hbm, px_hbm, py_hbm, rb_hbm, out_hbm, px_v, py_v, rb_v,
             i00, i10, i01, i11, wb, r00, r10, r01, r11, ov,
             s0, s1, s2, s3):
        wid = lax.axis_index("s")
        base = wid * PPW
        pltpu.sync_copy(px_hbm.at[pl.ds(base, PPW)], px_v)
        pltpu.sync_copy(py_hbm.at[pl.ds(base, PPW)], py_v)
        pltpu.sync_copy(rb_hbm.at[pl.ds(base, PPW)], rb_v)

        def chunk(g, carry):
            goff = g * G
            for j in range(G // L):
                off = goff + j * L
                px = px_v[pl.ds(off, L)]
                py = py_v[pl.ds(off, L)]
                gx = 2.0 * (px / float(W - 1)) - 1.0
                gy = 2.0 * (py / float(H - 1)) - 1.0
                ix = ((gx + 1.0) * float(W) - 1.0) / 2.0
                iy = ((gy + 1.0) * float(H) - 1.0) / 2.0
                x0i = (ix + 1.0).astype(jnp.int32) - 1
                y0i = (iy + 1.0).astype(jnp.int32) - 1
                x0f = x0i.astype(jnp.float32)
                y0f = y0i.astype(jnp.float32)
                wx1 = ix - x0f
                wx0 = 1.0 - wx1
                wy1 = iy - y0f
                wy0 = 1.0 - wy1
                vx0 = jnp.clip(x0f + 1.0, 0.0, 1.0)
                vx1 = jnp.clip(float(W) - (x0f + 1.0), 0.0, 1.0)
                vy0 = jnp.clip(y0f + 1.0, 0.0, 1.0)
                vy1 = jnp.clip(float(H) - (y0f + 1.0), 0.0, 1.0)
                w00 = wx0 * wy0 * (vx0 * vy0)
                w10 = wx1 * wy0 * (vx1 * vy0)
                w01 = wx0 * wy1 * (vx0 * vy1)
                w11 = wx1 * wy1 * (vx1 * vy1)
                xc0 = jnp.clip(x0i, 0, W - 1)
                xc1 = jnp.clip(x0i + 1, 0, W - 1)
                yc0 = jnp.clip(y0i, 0, H - 1) * W
                yc1 = jnp.clip(y0i + 1, 0, H - 1) * W
                rowbase = rb_v[pl.ds(off, L)]
                sl = pl.ds(j * L, L)
                i00[sl] = rowbase + yc0 + xc0
                i10[sl] = rowbase + yc0 + xc1
                i01[sl] = rowbase + yc1 + xc0
                i11[sl] = rowbase + yc1 + xc1
                wb[0, sl] = w00
                wb[1, sl] = w10
                wb[2, sl] = w01
                wb[3, sl] = w11

            c0 = pltpu.async_copy(xt_hbm.at[i00], r00, s0)
            c1 = pltpu.async_copy(xt_hbm.at[i10], r10, s1)
            c2 = pltpu.async_copy(xt_hbm.at[i01], r01, s2)
            c3 = pltpu.async_copy(xt_hbm.at[i11], r11, s3)
            c0.wait()
            c1.wait()
            c2.wait()
            c3.wait()

            def gbody(j, carry2):
                gsl = pl.ds(j * L, L)
                w00v = wb[0, gsl]
                w10v = wb[1, gsl]
                w01v = wb[2, gsl]
                w11v = wb[3, gsl]
                for t in range(L):
                    p = j * L + t
                    w0 = w00v[t]
                    w1 = w10v[t]
                    w2 = w01v[t]
                    w3 = w11v[t]
                    for cb in range(CB):
                        csl = pl.ds(cb * L, L)
                        acc = (r00[p, csl] * w0 + r10[p, csl] * w1
                               + r01[p, csl] * w2 + r11[p, csl] * w3)
                        ov[p, csl] = acc
                return carry2

            lax.fori_loop(0, G // L, gbody, 0)
            pltpu.sync_copy(ov, out_hbm.at[pl.ds(base + goff, G)])
            return carry

        lax.fori_loop(0, NCH, chunk, 0)

    return body(xt, posx, posy, rowb)


def kernel(x, pos, H, W):
    B, C, Hs, Ws = x.shape
    _, N, _ = pos.shape
    NW, G = 16, 128
    tot = B * N
    NP = ((tot + NW * G - 1) // (NW * G)) * (NW * G)

    xt = _transpose_to_rows(x)
    posx = jnp.pad(pos[..., 0].reshape(-1), (0, NP - tot))
    posy = jnp.pad(pos[..., 1].reshape(-1), (0, NP - tot))
    p = jnp.arange(NP, dtype=jnp.int32)
    rowb = (jnp.minimum(p, tot - 1) // N) * (Hs * Ws)
    out = _sc_sample(xt, posx, posy, rowb, B, C, Hs, Ws, N, NP)
    return out[:tot].reshape(B, N, C)


# speedup vs baseline: 1.1644x; 1.1644x over previous
import functools

import jax
import jax.numpy as jnp
from jax import lax
from jax.experimental import pallas as pl
from jax.experimental.pallas import tpu as pltpu
from jax.experimental.pallas import tpu_sc as plsc

_CP = 128


def _transpose_to_rows(x):
    B, C, H, W = x.shape
    HB = 16

    def body(x_ref, o_ref):
        xb = x_ref[0]
        xb = xb.reshape(C, HB * W)
        o_ref[...] = jnp.pad(xb.T, ((0, 0), (0, _CP - C)))

    return pl.pallas_call(
        body,
        grid=(B, H // HB),
        in_specs=[pl.BlockSpec((1, C, HB, W), lambda b, h: (b, 0, h, 0))],
        out_specs=pl.BlockSpec((HB * W, _CP), lambda b, h: (b * (H // HB) + h, 0)),
        out_shape=jax.ShapeDtypeStruct((B * H * W, _CP), x.dtype),
    )(x)


def _sc_sample(xt, posx, posy, rowb, B, C, H, W, N, NP):
    info = plsc.get_sparse_core_info()
    NC, NS, L = info.num_cores, info.num_subcores, info.num_lanes
    NW = NC * NS
    PPW = NP // NW
    G = 64
    NCH = PPW // G
    CB = C // L

    mesh = plsc.VectorSubcoreMesh(core_axis_name="c", subcore_axis_name="s")

    @functools.partial(
        pl.kernel, mesh=mesh,
        out_type=jax.ShapeDtypeStruct((NP, C), jnp.float32),
        scratch_types=[
            pltpu.VMEM((PPW,), jnp.float32),
            pltpu.VMEM((PPW,), jnp.float32),
            pltpu.VMEM((PPW,), jnp.int32),
            [pltpu.VMEM((4, G), jnp.int32)] * 2,
            [pltpu.VMEM((4, G), jnp.float32)] * 2,
            [[pltpu.VMEM((G, _CP), jnp.float32)] * 4] * 2,
            pltpu.VMEM((G, C), jnp.float32),
            [[pltpu.SemaphoreType.DMA] * 4] * 2,
        ],
    )
    def body(xt_hbm, px_hbm, py_hbm, rb_hbm, out_hbm, px_v, py_v, rb_v,
             idx2, wb2, rows2, ov, sem2):
        wid = lax.axis_index("c") * NS + lax.axis_index("s")
        base = wid * PPW
        pltpu.sync_copy(px_hbm.at[pl.ds(base, PPW)], px_v)
        pltpu.sync_copy(py_hbm.at[pl.ds(base, PPW)], py_v)
        pltpu.sync_copy(rb_hbm.at[pl.ds(base, PPW)], rb_v)

        def calc_idx(g, bi):
            idx = idx2[bi]
            wb = wb2[bi]
            goff = g * G
            for j in range(G // L):
                off = goff + j * L
                px = px_v[pl.ds(off, L)]
                py = py_v[pl.ds(off, L)]
                gx = 2.0 * (px / float(W - 1)) - 1.0
                gy = 2.0 * (py / float(H - 1)) - 1.0
                ix = ((gx + 1.0) * float(W) - 1.0) / 2.0
                iy = ((gy + 1.0) * float(H) - 1.0) / 2.0
                x0i = (ix + 1.0).astype(jnp.int32) - 1
                y0i = (iy + 1.0).astype(jnp.int32) - 1
                x0f = x0i.astype(jnp.float32)
                y0f = y0i.astype(jnp.float32)
                wx1 = ix - x0f
                wx0 = 1.0 - wx1
                wy1 = iy - y0f
                wy0 = 1.0 - wy1
                vx0 = jnp.clip(x0f + 1.0, 0.0, 1.0)
                vx1 = jnp.clip(float(W) - (x0f + 1.0), 0.0, 1.0)
                vy0 = jnp.clip(y0f + 1.0, 0.0, 1.0)
                vy1 = jnp.clip(float(H) - (y0f + 1.0), 0.0, 1.0)
                xc0 = jnp.clip(x0i, 0, W - 1)
                xc1 = jnp.clip(x0i + 1, 0, W - 1)
                yc0 = jnp.clip(y0i, 0, H - 1) * W
                yc1 = jnp.clip(y0i + 1, 0, H - 1) * W
                rowbase = rb_v[pl.ds(off, L)]
                sl = pl.ds(j * L, L)
                idx[0, sl] = rowbase + yc0 + xc0
                idx[1, sl] = rowbase + yc0 + xc1
                idx[2, sl] = rowbase + yc1 + xc0
                idx[3, sl] = rowbase + yc1 + xc1
                wb[0, sl] = wx0 * wy0 * (vx0 * vy0)
                wb[1, sl] = wx1 * wy0 * (vx1 * vy0)
                wb[2, sl] = wx0 * wy1 * (vx0 * vy1)
                wb[3, sl] = wx1 * wy1 * (vx1 * vy1)

        def fire(bi):
            for k in range(4):
                pltpu.async_copy(xt_hbm.at[idx2[bi].at[k]], rows2[bi][k],
                                 sem2[bi][k])

        def drain(bi):
            for k in range(4):
                pltpu.make_async_copy(xt_hbm.at[idx2[bi].at[k]], rows2[bi][k],
                                      sem2[bi][k]).wait()

        def combine(g, bi):
            rows = rows2[bi]
            wb = wb2[bi]

            def gbody(j, carry2):
                gsl = pl.ds(j * L, L)
                w00v = wb[0, gsl]
                w10v = wb[1, gsl]
                w01v = wb[2, gsl]
                w11v = wb[3, gsl]
                for t in range(L):
                    p = j * L + t
                    w0 = w00v[t]
                    w1 = w10v[t]
                    w2 = w01v[t]
                    w3 = w11v[t]
                    for cb in range(CB):
                        csl = pl.ds(cb * L, L)
                        acc = (rows[0][p, csl] * w0 + rows[1][p, csl] * w1
                               + rows[2][p, csl] * w2 + rows[3][p, csl] * w3)
                        ov[p, csl] = acc
                return carry2

            lax.fori_loop(0, G // L, gbody, 0)
            pltpu.sync_copy(ov, out_hbm.at[pl.ds(base + g * G, G)])

        calc_idx(0, 0)
        fire(0)

        def pair(g2, carry):
            g0 = g2 * 2

            calc_idx(g0 + 1, 1)
            fire(1)
            drain(0)
            combine(g0, 0)

            @pl.when(g2 < NCH // 2 - 1)
            def _():
                calc_idx(g0 + 2, 0)
                fire(0)

            drain(1)
            combine(g0 + 1, 1)
            return carry

        lax.fori_loop(0, NCH // 2, pair, 0)

    return body(xt, posx, posy, rowb)


def kernel(x, pos, H, W):
    B, C, Hs, Ws = x.shape
    _, N, _ = pos.shape
    NW, G = 32, 128
    tot = B * N
    NP = ((tot + NW * G - 1) // (NW * G)) * (NW * G)

    xt = _transpose_to_rows(x)
    posx = jnp.pad(pos[..., 0].reshape(-1), (0, NP - tot))
    posy = jnp.pad(pos[..., 1].reshape(-1), (0, NP - tot))
    p = jnp.arange(NP, dtype=jnp.int32)
    rowb = (jnp.minimum(p, tot - 1) // N) * (Hs * Ws)
    out = _sc_sample(xt, posx, posy, rowb, B, C, Hs, Ws, N, NP)
    return out[:tot].reshape(B, N, C)


# speedup vs baseline: 1.1653x; 1.0008x over previous
import functools

import jax
import jax.numpy as jnp
from jax import lax
from jax.experimental import pallas as pl
from jax.experimental.pallas import tpu as pltpu
from jax.experimental.pallas import tpu_sc as plsc

_CP = 128


def _transpose_to_rows(x):
    B, C, H, W = x.shape
    HB = 16

    def body(x_ref, o_ref):
        xb = x_ref[0]
        xb = xb.reshape(C, HB * W)
        o_ref[...] = jnp.pad(xb.T, ((0, 0), (0, _CP - C)))

    return pl.pallas_call(
        body,
        grid=(B, H // HB),
        in_specs=[pl.BlockSpec((1, C, HB, W), lambda b, h: (b, 0, h, 0))],
        out_specs=pl.BlockSpec((HB * W, _CP), lambda b, h: (b * (H // HB) + h, 0)),
        out_shape=jax.ShapeDtypeStruct((B * H * W, _CP), x.dtype),
    )(x)


def _sc_sample(xt, posx, posy, rowb, B, C, H, W, N, NP):
    info = plsc.get_sparse_core_info()
    NC, NS, L = info.num_cores, info.num_subcores, info.num_lanes
    NW = NC * NS
    PPW = NP // NW
    G = 64
    NCH = PPW // G
    CB = C // L

    mesh = plsc.VectorSubcoreMesh(core_axis_name="c", subcore_axis_name="s")

    @functools.partial(
        pl.kernel, mesh=mesh,
        out_type=jax.ShapeDtypeStruct((NP, C), jnp.float32),
        scratch_types=[
            pltpu.VMEM((PPW,), jnp.float32),
            pltpu.VMEM((PPW,), jnp.float32),
            pltpu.VMEM((PPW,), jnp.int32),
            [pltpu.VMEM((4, G), jnp.int32)] * 2,
            [pltpu.VMEM((4, G), jnp.float32)] * 2,
            [[pltpu.VMEM((G, _CP), jnp.float32)] * 4] * 2,
            pltpu.VMEM((G, C), jnp.float32),
            [[pltpu.SemaphoreType.DMA] * 4] * 2,
        ],
    )
    def body(xt_hbm, px_hbm, py_hbm, rb_hbm, out_hbm, px_v, py_v, rb_v,
             idx2, wb2, rows2, ov, sem2):
        wid = lax.axis_index("c") * NS + lax.axis_index("s")
        base = wid * PPW
        pltpu.sync_copy(px_hbm.at[pl.ds(base, PPW)], px_v)
        pltpu.sync_copy(py_hbm.at[pl.ds(base, PPW)], py_v)
        pltpu.sync_copy(rb_hbm.at[pl.ds(base, PPW)], rb_v)

        def calc_idx(g, bi):
            idx = idx2[bi]
            wb = wb2[bi]
            goff = g * G
            for j in range(G // L):
                off = goff + j * L
                px = px_v[pl.ds(off, L)]
                py = py_v[pl.ds(off, L)]
                gx = 2.0 * (px / float(W - 1)) - 1.0
                gy = 2.0 * (py / float(H - 1)) - 1.0
                ix = ((gx + 1.0) * float(W) - 1.0) / 2.0
                iy = ((gy + 1.0) * float(H) - 1.0) / 2.0
                x0i = (ix + 1.0).astype(jnp.int32) - 1
                y0i = (iy + 1.0).astype(jnp.int32) - 1
                x0f = x0i.astype(jnp.float32)
                y0f = y0i.astype(jnp.float32)
                wx1 = ix - x0f
                wx0 = 1.0 - wx1
                wy1 = iy - y0f
                wy0 = 1.0 - wy1
                vx0 = jnp.clip(x0f + 1.0, 0.0, 1.0)
                vx1 = jnp.clip(float(W) - (x0f + 1.0), 0.0, 1.0)
                vy0 = jnp.clip(y0f + 1.0, 0.0, 1.0)
                vy1 = jnp.clip(float(H) - (y0f + 1.0), 0.0, 1.0)
                xc0 = jnp.clip(x0i, 0, W - 1)
                xc1 = jnp.clip(x0i + 1, 0, W - 1)
                yc0 = jnp.clip(y0i, 0, H - 1) * W
                yc1 = jnp.clip(y0i + 1, 0, H - 1) * W
                rowbase = rb_v[pl.ds(off, L)]
                sl = pl.ds(j * L, L)
                idx[0, sl] = rowbase + yc0 + xc0
                idx[1, sl] = rowbase + yc0 + xc1
                idx[2, sl] = rowbase + yc1 + xc0
                idx[3, sl] = rowbase + yc1 + xc1
                wb[0, sl] = wx0 * wy0 * (vx0 * vy0)
                wb[1, sl] = wx1 * wy0 * (vx1 * vy0)
                wb[2, sl] = wx0 * wy1 * (vx0 * vy1)
                wb[3, sl] = wx1 * wy1 * (vx1 * vy1)

        SS = 2
        GS = G // SS

        def fire(bi):
            for k in range(4):
                for i in range(SS):
                    pltpu.async_copy(
                        xt_hbm.at[idx2[bi].at[k, pl.ds(i * GS, GS)]],
                        rows2[bi][k].at[pl.ds(i * GS, GS)],
                        sem2[bi][k])

        def drain(bi):
            for k in range(4):
                for i in range(SS):
                    pltpu.make_async_copy(
                        xt_hbm.at[idx2[bi].at[k, pl.ds(i * GS, GS)]],
                        rows2[bi][k].at[pl.ds(i * GS, GS)],
                        sem2[bi][k]).wait()

        def combine(g, bi):
            rows = rows2[bi]
            wb = wb2[bi]

            def gbody(j, carry2):
                gsl = pl.ds(j * L, L)
                w00v = wb[0, gsl]
                w10v = wb[1, gsl]
                w01v = wb[2, gsl]
                w11v = wb[3, gsl]
                for t in range(L):
                    p = j * L + t
                    w0 = w00v[t]
                    w1 = w10v[t]
                    w2 = w01v[t]
                    w3 = w11v[t]
                    for cb in range(CB):
                        csl = pl.ds(cb * L, L)
                        acc = (rows[0][p, csl] * w0 + rows[1][p, csl] * w1
                               + rows[2][p, csl] * w2 + rows[3][p, csl] * w3)
                        ov[p, csl] = acc
                return carry2

            lax.fori_loop(0, G // L, gbody, 0)
            pltpu.sync_copy(ov, out_hbm.at[pl.ds(base + g * G, G)])

        calc_idx(0, 0)
        fire(0)

        def pair(g2, carry):
            g0 = g2 * 2

            calc_idx(g0 + 1, 1)
            fire(1)
            drain(0)
            combine(g0, 0)

            @pl.when(g2 < NCH // 2 - 1)
            def _():
                calc_idx(g0 + 2, 0)
                fire(0)

            drain(1)
            combine(g0 + 1, 1)
            return carry

        lax.fori_loop(0, NCH // 2, pair, 0)

    return body(xt, posx, posy, rowb)


def kernel(x, pos, H, W):
    B, C, Hs, Ws = x.shape
    _, N, _ = pos.shape
    NW, G = 32, 128
    tot = B * N
    NP = ((tot + NW * G - 1) // (NW * G)) * (NW * G)

    xt = _transpose_to_rows(x)
    posx = jnp.pad(pos[..., 0].reshape(-1), (0, NP - tot))
    posy = jnp.pad(pos[..., 1].reshape(-1), (0, NP - tot))
    p = jnp.arange(NP, dtype=jnp.int32)
    rowb = (jnp.minimum(p, tot - 1) // N) * (Hs * Ws)
    out = _sc_sample(xt, posx, posy, rowb, B, C, Hs, Ws, N, NP)
    return out[:tot].reshape(B, N, C)
